# baseline (speedup 1.0000x reference)
import functools

import jax
import jax.numpy as jnp
from jax.experimental import pallas as pl
from jax.experimental.pallas import tpu as pltpu

QT = 1024
KB = 2000
TOPK = 20


def _topk_body(q_ref, k_ref, vals_ref, idx_ref,
               s_scr, sv_scr, si_scr, bm_scr, ts_scr, *, nkb):
    ph = pl.program_id(0)
    j = pl.program_id(1)

    q = q_ref[...]
    kb = k_ref[...]
    s = jax.lax.dot_general(kb, q, (((1,), (1,)), ((), ())),
                            preferred_element_type=jnp.float32)

    @pl.when(ph == 0)
    def _phase0():
        bm_scr[pl.ds(j, 1)] = jnp.max(s, axis=0, keepdims=True)[None]

    @pl.when(ph == 1)
    def _phase1():
        @pl.when(j == 0)
        def _seed():
            sv_scr[...] = jnp.full((TOPK, QT), -jnp.inf, dtype=jnp.float32)
            si_scr[...] = jnp.zeros((TOPK, QT), dtype=jnp.int32)
            brows = jax.lax.broadcasted_iota(jnp.int32, (nkb, QT), 0)
            bm0 = bm_scr[...].reshape(nkb, QT)

            def mask_max(_, bm):
                v = jnp.max(bm, axis=0, keepdims=True)
                c = jnp.min(jnp.where(bm == v, brows, nkb),
                            axis=0, keepdims=True)
                return jnp.where(brows == c, -jnp.inf, bm)

            bm = jax.lax.fori_loop(0, TOPK - 1, mask_max, bm0)
            ts_scr[...] = jnp.max(bm, axis=0, keepdims=True)

        s_scr[...] = s

        rows = jax.lax.broadcasted_iota(jnp.int32, (KB, QT), 0)
        io20 = jax.lax.broadcasted_iota(jnp.int32, (TOPK, QT), 0)

        rv0 = sv_scr[...]
        ri0 = si_scr[...]
        rm0 = bm_scr[pl.ds(j, 1)].reshape(1, QT)
        tseed = ts_scr[...]

        def cond(carry):
            cnt, rv, _ri, rm = carry
            live = (rm > rv[TOPK - 1:TOPK, :]) & (rm >= tseed)
            return jnp.logical_and(cnt < TOPK, jnp.any(live))

        def body(carry):
            cnt, rv, ri, rm = carry
            blk = s_scr[...]
            c = jnp.min(jnp.where(blk == rm, rows, KB),
                        axis=0, keepdims=True)
            blk = jnp.where(rows == c, -jnp.inf, blk)
            s_scr[...] = blk
            new_rm = jnp.max(blk, axis=0, keepdims=True)
            p = jnp.sum((rv >= rm).astype(jnp.int32), axis=0, keepdims=True)
            gi = c + j * KB
            rv_sh = jnp.concatenate([rv[:1, :], rv[:TOPK - 1, :]], axis=0)
            ri_sh = jnp.concatenate([ri[:1, :], ri[:TOPK - 1, :]], axis=0)
            rv = jnp.where(io20 < p, rv, jnp.where(io20 == p, rm, rv_sh))
            ri = jnp.where(io20 < p, ri, jnp.where(io20 == p, gi, ri_sh))
            return cnt + 1, rv, ri, new_rm

        _, rv, ri, _ = jax.lax.while_loop(
            cond, body, (jnp.int32(0), rv0, ri0, rm0))
        sv_scr[...] = rv
        si_scr[...] = ri

        @pl.when(j == nkb - 1)
        def _emit():
            vals_ref[...] = rv
            idx_ref[...] = ri


def kernel(queries, keys, k):
    nq, d = queries.shape
    nk, _ = keys.shape
    nkb = nk // KB

    vals_t, idx_t = pl.pallas_call(
        functools.partial(_topk_body, nkb=nkb),
        grid=(2, nkb),
        in_specs=[
            pl.BlockSpec((QT, d), lambda ph, j: (0, 0)),
            pl.BlockSpec((KB, d), lambda ph, j: (j, 0)),
        ],
        out_specs=[
            pl.BlockSpec((TOPK, QT), lambda ph, j: (0, 0)),
            pl.BlockSpec((TOPK, QT), lambda ph, j: (0, 0)),
        ],
        out_shape=[
            jax.ShapeDtypeStruct((TOPK, nq), jnp.float32),
            jax.ShapeDtypeStruct((TOPK, nq), jnp.int32),
        ],
        scratch_shapes=[
            pltpu.VMEM((KB, QT), jnp.float32),
            pltpu.VMEM((TOPK, QT), jnp.float32),
            pltpu.VMEM((TOPK, QT), jnp.int32),
            pltpu.VMEM((nkb, 1, QT), jnp.float32),
            pltpu.VMEM((1, QT), jnp.float32),
        ],
        compiler_params=pltpu.CompilerParams(
            dimension_semantics=("arbitrary", "arbitrary"),
        ),
    )(queries, keys)
    return (vals_t.T, idx_t.T + (k - TOPK))


# speedup vs baseline: 1.0802x; 1.0802x over previous
import functools

import jax
import jax.numpy as jnp
from jax.experimental import pallas as pl
from jax.experimental.pallas import tpu as pltpu

QT = 1024
KB = 1000
TOPK = 20


def _topk_body(q_ref, k_ref, vals_ref, idx_ref,
               s_scr, sv_scr, si_scr, bm_scr, ts_scr, *, nkb):
    ph = pl.program_id(0)
    j = pl.program_id(1)

    q = q_ref[...]
    kb = k_ref[...]
    s = jax.lax.dot_general(kb, q, (((1,), (1,)), ((), ())),
                            preferred_element_type=jnp.float32)

    @pl.when(ph == 0)
    def _phase0():
        bm_scr[pl.ds(j, 1)] = jnp.max(s, axis=0, keepdims=True)[None]

    @pl.when(ph == 1)
    def _phase1():
        @pl.when(j == 0)
        def _seed():
            sv_scr[...] = jnp.full((TOPK, QT), -jnp.inf, dtype=jnp.float32)
            si_scr[...] = jnp.zeros((TOPK, QT), dtype=jnp.int32)
            brows = jax.lax.broadcasted_iota(jnp.int32, (nkb, QT), 0)
            bm0 = bm_scr[...].reshape(nkb, QT)

            def mask_max(_, bm):
                v = jnp.max(bm, axis=0, keepdims=True)
                c = jnp.min(jnp.where(bm == v, brows, nkb),
                            axis=0, keepdims=True)
                return jnp.where(brows == c, -jnp.inf, bm)

            bm = jax.lax.fori_loop(0, TOPK - 1, mask_max, bm0)
            ts_scr[...] = jnp.max(bm, axis=0, keepdims=True)

        s_scr[...] = s

        rows = jax.lax.broadcasted_iota(jnp.int32, (KB, QT), 0)
        io20 = jax.lax.broadcasted_iota(jnp.int32, (TOPK, QT), 0)

        rv0 = sv_scr[...]
        ri0 = si_scr[...]
        rm0 = bm_scr[pl.ds(j, 1)].reshape(1, QT)
        tseed = ts_scr[...]

        def cond(carry):
            cnt, rv, _ri, rm = carry
            live = (rm > rv[TOPK - 1:TOPK, :]) & (rm >= tseed)
            return jnp.logical_and(cnt < TOPK, jnp.any(live))

        def body(carry):
            cnt, rv, ri, rm = carry
            blk = s_scr[...]
            c = jnp.min(jnp.where(blk == rm, rows, KB),
                        axis=0, keepdims=True)
            blk = jnp.where(rows == c, -jnp.inf, blk)
            s_scr[...] = blk
            new_rm = jnp.max(blk, axis=0, keepdims=True)
            p = jnp.sum((rv >= rm).astype(jnp.int32), axis=0, keepdims=True)
            gi = c + j * KB
            rv_sh = jnp.concatenate([rv[:1, :], rv[:TOPK - 1, :]], axis=0)
            ri_sh = jnp.concatenate([ri[:1, :], ri[:TOPK - 1, :]], axis=0)
            rv = jnp.where(io20 < p, rv, jnp.where(io20 == p, rm, rv_sh))
            ri = jnp.where(io20 < p, ri, jnp.where(io20 == p, gi, ri_sh))
            return cnt + 1, rv, ri, new_rm

        _, rv, ri, _ = jax.lax.while_loop(
            cond, body, (jnp.int32(0), rv0, ri0, rm0))
        sv_scr[...] = rv
        si_scr[...] = ri

        @pl.when(j == nkb - 1)
        def _emit():
            vals_ref[...] = rv
            idx_ref[...] = ri


def kernel(queries, keys, k):
    nq, d = queries.shape
    nk, _ = keys.shape
    nkb = nk // KB

    vals_t, idx_t = pl.pallas_call(
        functools.partial(_topk_body, nkb=nkb),
        grid=(2, nkb),
        in_specs=[
            pl.BlockSpec((QT, d), lambda ph, j: (0, 0)),
            pl.BlockSpec((KB, d), lambda ph, j: (j, 0)),
        ],
        out_specs=[
            pl.BlockSpec((TOPK, QT), lambda ph, j: (0, 0)),
            pl.BlockSpec((TOPK, QT), lambda ph, j: (0, 0)),
        ],
        out_shape=[
            jax.ShapeDtypeStruct((TOPK, nq), jnp.float32),
            jax.ShapeDtypeStruct((TOPK, nq), jnp.int32),
        ],
        scratch_shapes=[
            pltpu.VMEM((KB, QT), jnp.float32),
            pltpu.VMEM((TOPK, QT), jnp.float32),
            pltpu.VMEM((TOPK, QT), jnp.int32),
            pltpu.VMEM((nkb, 1, QT), jnp.float32),
            pltpu.VMEM((1, QT), jnp.float32),
        ],
        compiler_params=pltpu.CompilerParams(
            dimension_semantics=("arbitrary", "arbitrary"),
        ),
    )(queries, keys)
    return (vals_t.T, idx_t.T + (k - TOPK))


# speedup vs baseline: 1.0904x; 1.0095x over previous
import functools

import jax
import jax.numpy as jnp
from jax.experimental import pallas as pl
from jax.experimental.pallas import tpu as pltpu

QT = 1024
KB = 800
TOPK = 20


def _topk_body(q_ref, k_ref, vals_ref, idx_ref,
               s_scr, sv_scr, si_scr, bm_scr, ts_scr, *, nkb):
    ph = pl.program_id(0)
    j = pl.program_id(1)

    q = q_ref[...]
    kb = k_ref[...]
    s = jax.lax.dot_general(kb, q, (((1,), (1,)), ((), ())),
                            preferred_element_type=jnp.float32)

    @pl.when(ph == 0)
    def _phase0():
        bm_scr[pl.ds(j, 1)] = jnp.max(s, axis=0, keepdims=True)[None]

    @pl.when(ph == 1)
    def _phase1():
        @pl.when(j == 0)
        def _seed():
            sv_scr[...] = jnp.full((TOPK, QT), -jnp.inf, dtype=jnp.float32)
            si_scr[...] = jnp.zeros((TOPK, QT), dtype=jnp.int32)
            brows = jax.lax.broadcasted_iota(jnp.int32, (nkb, QT), 0)
            bm0 = bm_scr[...].reshape(nkb, QT)

            def mask_max(_, bm):
                v = jnp.max(bm, axis=0, keepdims=True)
                c = jnp.min(jnp.where(bm == v, brows, nkb),
                            axis=0, keepdims=True)
                return jnp.where(brows == c, -jnp.inf, bm)

            bm = jax.lax.fori_loop(0, TOPK - 1, mask_max, bm0)
            ts_scr[...] = jnp.max(bm, axis=0, keepdims=True)

        s_scr[...] = s

        rows = jax.lax.broadcasted_iota(jnp.int32, (KB, QT), 0)
        io20 = jax.lax.broadcasted_iota(jnp.int32, (TOPK, QT), 0)

        rv0 = sv_scr[...]
        ri0 = si_scr[...]
        rm0 = bm_scr[pl.ds(j, 1)].reshape(1, QT)
        tseed = ts_scr[...]

        def cond(carry):
            cnt, rv, _ri, rm = carry
            live = (rm > rv[TOPK - 1:TOPK, :]) & (rm >= tseed)
            return jnp.logical_and(cnt < TOPK, jnp.any(live))

        def body(carry):
            cnt, rv, ri, rm = carry
            blk = s_scr[...]
            c = jnp.min(jnp.where(blk == rm, rows, KB),
                        axis=0, keepdims=True)
            blk = jnp.where(rows == c, -jnp.inf, blk)
            s_scr[...] = blk
            new_rm = jnp.max(blk, axis=0, keepdims=True)
            p = jnp.sum((rv >= rm).astype(jnp.int32), axis=0, keepdims=True)
            gi = c + j * KB
            rv_sh = jnp.concatenate([rv[:1, :], rv[:TOPK - 1, :]], axis=0)
            ri_sh = jnp.concatenate([ri[:1, :], ri[:TOPK - 1, :]], axis=0)
            rv = jnp.where(io20 < p, rv, jnp.where(io20 == p, rm, rv_sh))
            ri = jnp.where(io20 < p, ri, jnp.where(io20 == p, gi, ri_sh))
            return cnt + 1, rv, ri, new_rm

        _, rv, ri, _ = jax.lax.while_loop(
            cond, body, (jnp.int32(0), rv0, ri0, rm0))
        sv_scr[...] = rv
        si_scr[...] = ri

        @pl.when(j == nkb - 1)
        def _emit():
            vals_ref[...] = rv
            idx_ref[...] = ri


def kernel(queries, keys, k):
    nq, d = queries.shape
    nk, _ = keys.shape
    nkb = nk // KB

    vals_t, idx_t = pl.pallas_call(
        functools.partial(_topk_body, nkb=nkb),
        grid=(2, nkb),
        in_specs=[
            pl.BlockSpec((QT, d), lambda ph, j: (0, 0)),
            pl.BlockSpec((KB, d), lambda ph, j: (j, 0)),
        ],
        out_specs=[
            pl.BlockSpec((TOPK, QT), lambda ph, j: (0, 0)),
            pl.BlockSpec((TOPK, QT), lambda ph, j: (0, 0)),
        ],
        out_shape=[
            jax.ShapeDtypeStruct((TOPK, nq), jnp.float32),
            jax.ShapeDtypeStruct((TOPK, nq), jnp.int32),
        ],
        scratch_shapes=[
            pltpu.VMEM((KB, QT), jnp.float32),
            pltpu.VMEM((TOPK, QT), jnp.float32),
            pltpu.VMEM((TOPK, QT), jnp.int32),
            pltpu.VMEM((nkb, 1, QT), jnp.float32),
            pltpu.VMEM((1, QT), jnp.float32),
        ],
        compiler_params=pltpu.CompilerParams(
            dimension_semantics=("arbitrary", "arbitrary"),
        ),
    )(queries, keys)
    return (vals_t.T, idx_t.T + (k - TOPK))
